# baseline (speedup 1.0000x reference)
import functools

import jax
import jax.numpy as jnp
from jax import lax
from jax.experimental import pallas as pl
from jax.experimental.pallas import tpu as pltpu
from jax.experimental.pallas import tpu_sc as plsc

F32 = jnp.float32
I32 = jnp.int32

_N = 10000
_E = 320000
_C = 20000
_SE = 100000
_G = 128
_H = 128
_HH = 64

_NC = 2
_TAB_R = 81920
_NS = 16

_EP = 327680
_SEP = 114688
_CG = 20480
_NG = 10240


def _sds(shape, dtype):
    return jax.ShapeDtypeStruct(shape, dtype)


def _node_tables(x, cWn, cbn, cWm, cWo, tWn, tbn, tWm, tWo):
    R = 512
    nb = pl.cdiv(_N, R)

    def body(x_r, cWn_r, cbn_r, cWm_r, cWo_r, tWn_r, tbn_r, tWm_r, tWo_r,
             hmc_r, hoc_r, hmt_r, hot_r):
        xb = x_r[...]
        hc = jnp.maximum(
            jnp.dot(xb, cWn_r[...], preferred_element_type=F32) + cbn_r[...], 0.0)
        hmc = jnp.dot(hc, cWm_r[...], preferred_element_type=F32)
        hmc_r[0] = hmc[:, :_HH]
        hmc_r[1] = hmc[:, _HH:]
        hoc_r[...] = jnp.dot(hc, cWo_r[...], preferred_element_type=F32)
        ht = jnp.maximum(
            jnp.dot(xb, tWn_r[...], preferred_element_type=F32) + tbn_r[...], 0.0)
        hmt = jnp.dot(ht, tWm_r[...], preferred_element_type=F32)
        hmt_r[0] = hmt[:, :_HH]
        hmt_r[1] = hmt[:, _HH:]
        hot_r[...] = jnp.dot(ht, tWo_r[...], preferred_element_type=F32)

    wsp = pl.BlockSpec((_H, _H), lambda i: (0, 0))
    bsp = pl.BlockSpec((1, _H), lambda i: (0, 0))
    return pl.pallas_call(
        body,
        grid=(nb,),
        in_specs=[pl.BlockSpec((R, _H), lambda i: (i, 0)),
                  wsp, bsp, wsp, wsp, wsp, bsp, wsp, wsp],
        out_specs=[pl.BlockSpec((2, R, _HH), lambda i: (0, i, 0)),
                   pl.BlockSpec((R, _H), lambda i: (i, 0)),
                   pl.BlockSpec((2, R, _HH), lambda i: (0, i, 0)),
                   pl.BlockSpec((R, _H), lambda i: (i, 0))],
        out_shape=[_sds((2, _TAB_R, _HH), F32), _sds((_N, _H), F32),
                   _sds((2, _TAB_R, _HH), F32), _sds((_N, _H), F32)],
    )(x, cWn, cbn.reshape(1, _H), cWm, cWo, tWn, tbn.reshape(1, _H), tWm, tWo)


def _edge_tables(ea_pad, We, be, Wm, bm):
    M = ea_pad.shape[0]
    R = 1024
    nb = M // R

    def body(ea_r, We_r, be_r, Wm_r, bm_r, em_r):
        e = jnp.maximum(
            jnp.dot(ea_r[...], We_r[...], preferred_element_type=F32) + be_r[...],
            0.0)
        em = jnp.dot(e, Wm_r[...], preferred_element_type=F32) + bm_r[...]
        em_r[0] = em[:, :_HH]
        em_r[1] = em[:, _HH:]

    return pl.pallas_call(
        body,
        grid=(nb,),
        in_specs=[pl.BlockSpec((R, 16), lambda i: (i, 0)),
                  pl.BlockSpec((16, _H), lambda i: (0, 0)),
                  pl.BlockSpec((1, _H), lambda i: (0, 0)),
                  pl.BlockSpec((_H, _H), lambda i: (0, 0)),
                  pl.BlockSpec((1, _H), lambda i: (0, 0))],
        out_specs=[pl.BlockSpec((2, R, _HH), lambda i: (0, i, 0))],
        out_shape=[_sds((2, M, _HH), F32)],
    )(ea_pad, We, be.reshape(1, _H), Wm, bm.reshape(1, _H))[0]


def _sc_gather(cnm_pad, tnm_pad, sem_pad, ew, maskf_pad, batch, nw, hoc, ea):
    mesh = plsc.VectorSubcoreMesh(core_axis_name="c", subcore_axis_name="s",
                                  num_cores=_NC, num_subcores=_NS)

    @functools.partial(
        pl.kernel,
        out_type=[_sds((_SEP, 16), F32), _sds((_SEP,), F32),
                  _sds((_CG,), I32), _sds((_CG,), F32),
                  _sds((_CG, _H), F32), _sds((_NG,), I32)],
        mesh=mesh,
        compiler_params=pltpu.CompilerParams(use_tc_tiling_on_sc=False),
        scratch_types=[
            pltpu.VMEM((512,), I32),
            pltpu.VMEM((512, 16), F32),
            pltpu.VMEM((512,), F32),
            pltpu.VMEM((512,), F32),
            pltpu.VMEM((320,), I32),
            pltpu.VMEM((320, _H), F32),
            pltpu.VMEM((320,), I32),
            pltpu.VMEM((320,), F32),
            pltpu.SemaphoreType.DMA,
        ],
    )
    def k(cnm_h, tnm_h, sem_h, ew_h, maskf_h, batch_h, nw_h, hoc_h, ea_h,
          eac_o, ewc_o, segc_o, nwc_o, hocg_o, segt_o,
          semb, eab, ewb, maskb, idxb, hocb, segb, nwb, dsem):
        wid = lax.axis_index("s") * _NC + lax.axis_index("c")

        for kk in range(7):
            base = (wid * 7 + kk) * 512
            pltpu.sync_copy(sem_h.at[pl.ds(base, 512)], semb)
            pltpu.async_copy(ea_h.at[semb], eab, dsem).wait()
            pltpu.sync_copy(eab, eac_o.at[pl.ds(base, 512)])
            pltpu.async_copy(ew_h.at[semb], ewb, dsem).wait()
            pltpu.sync_copy(maskf_h.at[pl.ds(base, 512)], maskb)
            for j in range(32):
                sl = pl.ds(j * 16, 16)
                ewb[sl] = ewb[sl] * maskb[sl]
            pltpu.sync_copy(ewb, ewc_o.at[pl.ds(base, 512)])

        for kk in range(2):
            base = (wid * 2 + kk) * 320
            pltpu.sync_copy(cnm_h.at[pl.ds(base, 320)], idxb)
            pltpu.async_copy(hoc_h.at[idxb], hocb, dsem).wait()
            pltpu.sync_copy(hocb, hocg_o.at[pl.ds(base, 320)])
            pltpu.async_copy(batch_h.at[idxb], segb, dsem).wait()
            pltpu.async_copy(nw_h.at[idxb], nwb, dsem).wait()
            pltpu.sync_copy(segb, segc_o.at[pl.ds(base, 320)])
            pltpu.sync_copy(nwb, nwc_o.at[pl.ds(base, 320)])

        base = wid * 320
        pltpu.sync_copy(tnm_h.at[pl.ds(base, 320)], idxb)
        pltpu.async_copy(batch_h.at[idxb], segb, dsem).wait()
        pltpu.sync_copy(segb, segt_o.at[pl.ds(base, 320)])

    return k(cnm_pad, tnm_pad, sem_pad, ew, maskf_pad, batch, nw, hoc, ea)


_ACC_R = 7424
_VAL_R = 7168
_DUMMY = 7420
_CB = 256


def _sc_msg(hmt, emt, src_t, dst_t, ew_t, hmc, emc, src_c, dst_c, ew_c, cnm):
    mesh = plsc.VectorSubcoreMesh(core_axis_name="c", subcore_axis_name="s",
                                  num_cores=_NC, num_subcores=_NS)
    zz = _ACC_R // _NS
    zw = _VAL_R // _NS

    def _pairs(total):
        out, o = [], 0
        while o < total:
            sz = min(_CB, total - o)
            out.append((o, sz))
            o += sz
        return out

    pairs_z = _pairs(zz)
    pairs_w = _pairs(zw)

    scratch = [
        pltpu.VMEM((_CB,), I32),
        pltpu.VMEM((_CB,), I32),
        pltpu.VMEM((_CB,), I32),
        pltpu.VMEM((_CB,), F32),
        pltpu.VMEM((_CB, _HH), F32),
        pltpu.VMEM((_CB, _HH), F32),
        pltpu.VMEM((_CB, _HH), F32),
        pltpu.VMEM_SHARED((_ACC_R, _HH), F32),
        pltpu.SemaphoreType.DMA,
    ]

    @functools.partial(
        pl.kernel,
        out_type=[_sds((4 * _VAL_R, _HH), F32),
                  _sds((6 * _VAL_R, _HH), F32)],
        mesh=mesh, scratch_types=scratch,
        compiler_params=pltpu.CompilerParams(use_tc_tiling_on_sc=False))
    def k(hmt_h, emt_h, srct_h, dstt_h, ewt_h,
          hmc_h, emc_h, srcc_h, dstc_h, ewc_h, cnm_h,
          outt_h, outc_h,
          sidx, eidx, didx, ewbv, rows, emb, mb, acc, dsem):
        c = lax.axis_index("c")
        s = lax.axis_index("s")

        def zb(i, _):
            for g in range(4):
                mb[i, pl.ds(g * 16, 16)] = jnp.zeros((16,), F32)
            return 0

        def phase(hm_h, em_h, src_h, dst_h, ew_h, out_h, ep, nchunks,
                  npass, compose):
            ec = ep // _NS
            for p in range(npass):
                lax.fori_loop(0, _CB, zb, 0)
                for (off, sz) in pairs_z:
                    pltpu.sync_copy(mb.at[pl.ds(0, sz)],
                                    acc.at[pl.ds(s * zz + off, sz)])
                plsc.subcore_barrier()

                def chunk(kk, _):
                    base = s * ec + kk * _CB
                    pltpu.sync_copy(src_h.at[pl.ds(base, _CB)], sidx)
                    coff = c * _TAB_R
                    if compose:
                        pltpu.async_copy(cnm_h.at[sidx], eidx, dsem).wait()
                        for j in range(_CB // 16):
                            sl = pl.ds(j * 16, 16)
                            eidx[sl] = eidx[sl] + coff
                    else:
                        for j in range(_CB // 16):
                            sl = pl.ds(j * 16, 16)
                            eidx[sl] = sidx[sl] + coff
                    pltpu.async_copy(hm_h.at[eidx], rows, dsem).wait()
                    pltpu.sync_copy(em_h.at[pl.ds(c * ep + base, _CB)], emb)
                    pltpu.sync_copy(ew_h.at[pl.ds(base, _CB)], ewbv)
                    pltpu.sync_copy(dst_h.at[pl.ds(base, _CB)], didx)
                    if npass > 1:
                        for j in range(_CB // 16):
                            sl = pl.ds(j * 16, 16)
                            rel = didx[sl] - p * _VAL_R
                            ok = (rel >= 0) & (rel < _VAL_R)
                            didx[sl] = jnp.where(ok, rel, _DUMMY)

                    def ebody(j, _):
                        wv = ewbv[pl.ds(j * 16, 16)]
                        for e in range(16):
                            i = j * 16 + e
                            w = wv[e]
                            for g in range(4):
                                sl = pl.ds(g * 16, 16)
                                mb[i, sl] = jnp.maximum(
                                    rows[i, sl] + emb[i, sl], 0.0) * w
                        return 0
                    lax.fori_loop(0, _CB // 16, ebody, 0)
                    pltpu.sync_copy(mb, acc.at[didx], add=True)
                    return 0
                lax.fori_loop(0, nchunks, chunk, 0)

                plsc.subcore_barrier()
                for (off, sz) in pairs_w:
                    pltpu.sync_copy(
                        acc.at[pl.ds(s * zw + off, sz)],
                        out_h.at[pl.ds((c * npass + p) * _VAL_R
                                       + s * zw + off, sz)])
                plsc.subcore_barrier()

        phase(hmt_h, emt_h, srct_h, dstt_h, ewt_h, outt_h,
              _EP, _EP // _NS // _CB, 2, False)
        phase(hmc_h, emc_h, srcc_h, dstc_h, ewc_h, outc_h,
              _SEP, _SEP // _NS // _CB, 3, True)

    return k(hmt, emt, src_t, dst_t, ew_t, hmc, emc, src_c, dst_c, ew_c, cnm)


def _out_pool(ho, agg2, Wo, bo, nw2, seg2, m_valid):
    M = ho.shape[0]
    R = 512
    nb = pl.cdiv(M, R)

    def body(ho_r, agg_r, Wo_r, bo_r, nw_r, seg_r, emb_r, sum_s, cnt_s):
        i = pl.program_id(0)

        @pl.when(i == 0)
        def _():
            sum_s[...] = jnp.zeros((_G, _H), F32)
            cnt_s[...] = jnp.zeros((_G, _H), F32)

        Wof = Wo_r[...]
        aggO = (jnp.dot(agg_r[0], Wof[:_HH, :], preferred_element_type=F32) +
                jnp.dot(agg_r[1], Wof[_HH:, :], preferred_element_type=F32))
        o = jnp.maximum(ho_r[...] + aggO + bo_r[...], 0.0) * nw_r[...]
        sb = seg_r[...]
        gid = lax.broadcasted_iota(I32, (R, _G), 1)
        rid = lax.broadcasted_iota(I32, (R, _G), 0) + i * R
        oh = jnp.where((sb == gid) & (rid < m_valid), 1.0, 0.0).astype(F32)
        dn = (((0,), (0,)), ((), ()))
        sum_s[...] += lax.dot_general(oh, o, dn, preferred_element_type=F32)
        cnt_s[...] += lax.dot_general(oh, jnp.ones((R, _H), F32), dn,
                                      preferred_element_type=F32)

        @pl.when(i == nb - 1)
        def _():
            emb_r[...] = sum_s[...] / jnp.maximum(cnt_s[...], 1.0)

    return pl.pallas_call(
        body,
        grid=(nb,),
        in_specs=[pl.BlockSpec((R, _H), lambda i: (i, 0)),
                  pl.BlockSpec((2, R, _HH), lambda i: (0, i, 0)),
                  pl.BlockSpec((_H, _H), lambda i: (0, 0)),
                  pl.BlockSpec((1, _H), lambda i: (0, 0)),
                  pl.BlockSpec((R, 1), lambda i: (i, 0)),
                  pl.BlockSpec((R, 1), lambda i: (i, 0))],
        out_specs=[pl.BlockSpec((_G, _H), lambda i: (0, 0))],
        out_shape=[_sds((_G, _H), F32)],
        scratch_shapes=[pltpu.VMEM((_G, _H), F32), pltpu.VMEM((_G, _H), F32)],
    )(ho, agg2, Wo, bo.reshape(1, _H), nw2, seg2)[0]


def kernel(x, node_weight, edge_index, edge_attr, edge_weight, batch,
           context_nodes_mapper, target_nodes_mapper, combined_subgraphs,
           subgraphs_edges_mapper, context_edges_mask,
           ctx_W_node, ctx_b_node, ctx_W_edge, ctx_b_edge, ctx_W_msg, ctx_b_msg,
           ctx_W_out, ctx_b_out,
           tgt_W_node, tgt_b_node, tgt_W_edge, tgt_b_edge, tgt_W_msg, tgt_b_msg,
           tgt_W_out, tgt_b_out):
    src_t = jnp.pad(edge_index[0], (0, _EP - _E))
    dst_t = jnp.pad(edge_index[1], (0, _EP - _E))
    ew_t = jnp.pad(edge_weight, (0, _EP - _E))
    ea_t = jnp.pad(edge_attr, ((0, _EP - _E), (0, 0)))
    src_c = jnp.pad(combined_subgraphs[0], (0, _SEP - _SE))
    dst_c = jnp.pad(combined_subgraphs[1], (0, _SEP - _SE))
    sem_p = jnp.pad(subgraphs_edges_mapper, (0, _SEP - _SE))
    maskf = jnp.pad(context_edges_mask.astype(F32), (0, _SEP - _SE))
    cnm_p = jnp.pad(context_nodes_mapper, (0, _CG - _C))
    tnm_p = jnp.pad(target_nodes_mapper, (0, _NG - _N))

    hmc2, hoc, hmt2, hot = _node_tables(
        x, ctx_W_node, ctx_b_node, ctx_W_msg, ctx_W_out,
        tgt_W_node, tgt_b_node, tgt_W_msg, tgt_W_out)
    emt = _edge_tables(ea_t, tgt_W_edge, tgt_b_edge, tgt_W_msg, tgt_b_msg)

    ea_c, ew_c, segc, nw_c, hoc_g, segt = _sc_gather(
        cnm_p, tnm_p, sem_p, edge_weight, maskf, batch, node_weight, hoc,
        edge_attr)

    emc = _edge_tables(ea_c, ctx_W_edge, ctx_b_edge, ctx_W_msg, ctx_b_msg)

    agg_t, agg_c = _sc_msg(
        hmt2.reshape(2 * _TAB_R, _HH), emt.reshape(2 * _EP, _HH),
        src_t, dst_t, ew_t,
        hmc2.reshape(2 * _TAB_R, _HH), emc.reshape(2 * _SEP, _HH),
        src_c, dst_c, ew_c, context_nodes_mapper)

    emb_t = _out_pool(hot, agg_t.reshape(2, 2 * _VAL_R, _HH), tgt_W_out,
                      tgt_b_out, node_weight.reshape(_N, 1),
                      segt[:_N].reshape(_N, 1), _N)
    emb_c = _out_pool(hoc_g, agg_c.reshape(2, 3 * _VAL_R, _HH), ctx_W_out,
                      ctx_b_out, nw_c.reshape(_CG, 1),
                      segc.reshape(_CG, 1), _C)
    return jnp.stack([emb_c, emb_t])


# speedup vs baseline: 1.2304x; 1.2304x over previous
import functools

import jax
import jax.numpy as jnp
from jax import lax
from jax.experimental import pallas as pl
from jax.experimental.pallas import tpu as pltpu
from jax.experimental.pallas import tpu_sc as plsc

F32 = jnp.float32
I32 = jnp.int32

_N = 10000
_E = 320000
_C = 20000
_SE = 100000
_G = 128
_H = 128
_HH = 64

_NC = 2
_TAB_R = 81920
_NS = 16

_EP = 327680
_SEP = 114688
_CG = 20480
_NG = 10240


def _sds(shape, dtype):
    return jax.ShapeDtypeStruct(shape, dtype)


def _node_tables(x, cWn, cbn, cWm, cWo, tWn, tbn, tWm, tWo):
    R = 512
    nb = pl.cdiv(_N, R)

    def body(x_r, cWn_r, cbn_r, cWm_r, cWo_r, tWn_r, tbn_r, tWm_r, tWo_r,
             hmc_r, hoc_r, hmt_r, hot_r):
        xb = x_r[...]
        hc = jnp.maximum(
            jnp.dot(xb, cWn_r[...], preferred_element_type=F32) + cbn_r[...], 0.0)
        hmc = jnp.dot(hc, cWm_r[...], preferred_element_type=F32)
        hmc_r[0] = hmc[:, :_HH]
        hmc_r[1] = hmc[:, _HH:]
        hoc_r[...] = jnp.dot(hc, cWo_r[...], preferred_element_type=F32)
        ht = jnp.maximum(
            jnp.dot(xb, tWn_r[...], preferred_element_type=F32) + tbn_r[...], 0.0)
        hmt = jnp.dot(ht, tWm_r[...], preferred_element_type=F32)
        hmt_r[0] = hmt[:, :_HH]
        hmt_r[1] = hmt[:, _HH:]
        hot_r[...] = jnp.dot(ht, tWo_r[...], preferred_element_type=F32)

    wsp = pl.BlockSpec((_H, _H), lambda i: (0, 0))
    bsp = pl.BlockSpec((1, _H), lambda i: (0, 0))
    return pl.pallas_call(
        body,
        grid=(nb,),
        in_specs=[pl.BlockSpec((R, _H), lambda i: (i, 0)),
                  wsp, bsp, wsp, wsp, wsp, bsp, wsp, wsp],
        out_specs=[pl.BlockSpec((2, R, _HH), lambda i: (0, i, 0)),
                   pl.BlockSpec((R, _H), lambda i: (i, 0)),
                   pl.BlockSpec((2, R, _HH), lambda i: (0, i, 0)),
                   pl.BlockSpec((R, _H), lambda i: (i, 0))],
        out_shape=[_sds((2, _TAB_R, _HH), F32), _sds((_N, _H), F32),
                   _sds((2, _TAB_R, _HH), F32), _sds((_N, _H), F32)],
    )(x, cWn, cbn.reshape(1, _H), cWm, cWo, tWn, tbn.reshape(1, _H), tWm, tWo)


def _edge_tables(ea_pad, We, be, Wm, bm):
    M = ea_pad.shape[0]
    R = 1024
    nb = M // R

    def body(ea_r, We_r, be_r, Wm_r, bm_r, em_r):
        e = jnp.maximum(
            jnp.dot(ea_r[...], We_r[...], preferred_element_type=F32) + be_r[...],
            0.0)
        em = jnp.dot(e, Wm_r[...], preferred_element_type=F32) + bm_r[...]
        em_r[0] = em[:, :_HH]
        em_r[1] = em[:, _HH:]

    return pl.pallas_call(
        body,
        grid=(nb,),
        in_specs=[pl.BlockSpec((R, 16), lambda i: (i, 0)),
                  pl.BlockSpec((16, _H), lambda i: (0, 0)),
                  pl.BlockSpec((1, _H), lambda i: (0, 0)),
                  pl.BlockSpec((_H, _H), lambda i: (0, 0)),
                  pl.BlockSpec((1, _H), lambda i: (0, 0))],
        out_specs=[pl.BlockSpec((2, R, _HH), lambda i: (0, i, 0))],
        out_shape=[_sds((2, M, _HH), F32)],
    )(ea_pad, We, be.reshape(1, _H), Wm, bm.reshape(1, _H))[0]


def _sc_gather(cnm_pad, tnm_pad, sem_pad, srcc_pad, ew, maskf_pad, batch, nw,
               hoc, ea):
    mesh = plsc.VectorSubcoreMesh(core_axis_name="c", subcore_axis_name="s",
                                  num_cores=_NC, num_subcores=_NS)

    @functools.partial(
        pl.kernel,
        out_type=[_sds((_SEP, 16), F32), _sds((_SEP,), F32),
                  _sds((_SEP,), I32),
                  _sds((_CG,), I32), _sds((_CG,), F32),
                  _sds((_CG, _H), F32), _sds((_NG,), I32)],
        mesh=mesh,
        compiler_params=pltpu.CompilerParams(use_tc_tiling_on_sc=False),
        scratch_types=[
            pltpu.VMEM((512,), I32),
            pltpu.VMEM((512,), I32),
            pltpu.VMEM((512,), I32),
            pltpu.VMEM((512, 16), F32),
            pltpu.VMEM((512,), F32),
            pltpu.VMEM((512,), F32),
            pltpu.VMEM((320,), I32),
            pltpu.VMEM((320, _H), F32),
            pltpu.VMEM((320,), I32),
            pltpu.VMEM((320,), F32),
            pltpu.SemaphoreType.DMA,
        ],
    )
    def k(cnm_h, tnm_h, sem_h, srcc_h, ew_h, maskf_h, batch_h, nw_h, hoc_h,
          ea_h,
          eac_o, ewc_o, idxc_o, segc_o, nwc_o, hocg_o, segt_o,
          semb, srcb, cmpb, eab, ewb, maskb, idxb, hocb, segb, nwb, dsem):
        wid = lax.axis_index("s") * _NC + lax.axis_index("c")

        for kk in range(7):
            base = (wid * 7 + kk) * 512
            pltpu.sync_copy(sem_h.at[pl.ds(base, 512)], semb)
            pltpu.sync_copy(srcc_h.at[pl.ds(base, 512)], srcb)
            pltpu.async_copy(ea_h.at[semb], eab, dsem).wait()
            pltpu.sync_copy(eab, eac_o.at[pl.ds(base, 512)])
            pltpu.async_copy(cnm_h.at[srcb], cmpb, dsem).wait()
            pltpu.sync_copy(cmpb, idxc_o.at[pl.ds(base, 512)])
            pltpu.async_copy(ew_h.at[semb], ewb, dsem).wait()
            pltpu.sync_copy(maskf_h.at[pl.ds(base, 512)], maskb)
            for j in range(32):
                sl = pl.ds(j * 16, 16)
                ewb[sl] = ewb[sl] * maskb[sl]
            pltpu.sync_copy(ewb, ewc_o.at[pl.ds(base, 512)])

        for kk in range(2):
            base = (wid * 2 + kk) * 320
            pltpu.sync_copy(cnm_h.at[pl.ds(base, 320)], idxb)
            pltpu.async_copy(hoc_h.at[idxb], hocb, dsem).wait()
            pltpu.sync_copy(hocb, hocg_o.at[pl.ds(base, 320)])
            pltpu.async_copy(batch_h.at[idxb], segb, dsem).wait()
            pltpu.async_copy(nw_h.at[idxb], nwb, dsem).wait()
            pltpu.sync_copy(segb, segc_o.at[pl.ds(base, 320)])
            pltpu.sync_copy(nwb, nwc_o.at[pl.ds(base, 320)])

        base = wid * 320
        pltpu.sync_copy(tnm_h.at[pl.ds(base, 320)], idxb)
        pltpu.async_copy(batch_h.at[idxb], segb, dsem).wait()
        pltpu.sync_copy(segb, segt_o.at[pl.ds(base, 320)])

    return k(cnm_pad, tnm_pad, sem_pad, srcc_pad, ew, maskf_pad, batch, nw,
             hoc, ea)


_ACC_R = 7424
_VAL_R = 7168
_DUMMY = 7420
_CB = 256


def _sc_msg(hmt, emt, src_t, dst_t, ew_t, hmc, emc, src_c, dst_c, ew_c):
    mesh = plsc.VectorSubcoreMesh(core_axis_name="c", subcore_axis_name="s",
                                  num_cores=_NC, num_subcores=_NS)
    zz = _ACC_R // _NS
    zw = _VAL_R // _NS

    def _pairs(total):
        out, o = [], 0
        while o < total:
            sz = min(_CB, total - o)
            out.append((o, sz))
            o += sz
        return out

    pairs_z = _pairs(zz)
    pairs_w = _pairs(zw)

    nset = 2
    scratch = [
        [pltpu.VMEM((_CB,), I32) for _ in range(nset)],
        [pltpu.VMEM((_CB,), I32) for _ in range(nset)],
        [pltpu.VMEM((_CB,), F32) for _ in range(nset)],
        [pltpu.VMEM((_CB, _HH), F32) for _ in range(nset)],
        [pltpu.VMEM((_CB, _HH), F32) for _ in range(nset)],
        [pltpu.SemaphoreType.DMA for _ in range(nset)],
        [pltpu.SemaphoreType.DMA for _ in range(nset)],
        pltpu.VMEM_SHARED((_ACC_R, _HH), F32),
    ]

    @functools.partial(
        pl.kernel,
        out_type=[_sds((4 * _VAL_R, _HH), F32),
                  _sds((6 * _VAL_R, _HH), F32)],
        mesh=mesh, scratch_types=scratch,
        compiler_params=pltpu.CompilerParams(use_tc_tiling_on_sc=False))
    def k(hmt_h, emt_h, srct_h, dstt_h, ewt_h,
          hmc_h, emc_h, srcc_h, dstc_h, ewc_h,
          outt_h, outc_h,
          sidx, didx, ewbv, rows, emb, lsem, gsem, acc):
        c = lax.axis_index("c")
        s = lax.axis_index("s")

        def phase(hm_h, em_h, src_h, dst_h, ew_h, out_h, ep, nchunks, npass):
            ec = ep // _NS

            def issue_loads(b, kk):
                base = s * ec + kk * _CB
                pltpu.async_copy(src_h.at[pl.ds(base, _CB)], sidx[b], lsem[b])
                pltpu.async_copy(dst_h.at[pl.ds(base, _CB)], didx[b], lsem[b])
                pltpu.async_copy(ew_h.at[pl.ds(base, _CB)], ewbv[b], lsem[b])
                pltpu.async_copy(em_h.at[pl.ds(c * ep + base, _CB)], emb[b],
                                 lsem[b])

            def wait_loads(b, kk):
                base = s * ec + kk * _CB
                pltpu.make_async_copy(src_h.at[pl.ds(base, _CB)], sidx[b],
                                      lsem[b]).wait()
                pltpu.make_async_copy(dst_h.at[pl.ds(base, _CB)], didx[b],
                                      lsem[b]).wait()
                pltpu.make_async_copy(ew_h.at[pl.ds(base, _CB)], ewbv[b],
                                      lsem[b]).wait()
                pltpu.make_async_copy(em_h.at[pl.ds(c * ep + base, _CB)],
                                      emb[b], lsem[b]).wait()

            def process(b, p):
                coff = c * _TAB_R
                for j in range(_CB // 16):
                    sl = pl.ds(j * 16, 16)
                    sidx[b][sl] = sidx[b][sl] + coff
                gd = pltpu.async_copy(hm_h.at[sidx[b]], rows[b], gsem[b])
                if npass > 1:
                    for j in range(_CB // 16):
                        sl = pl.ds(j * 16, 16)
                        rel = didx[b][sl] - p * _VAL_R
                        ok = (rel >= 0) & (rel < _VAL_R)
                        didx[b][sl] = jnp.where(ok, rel, _DUMMY)
                gd.wait()

                def ebody(j, _):
                    wv = ewbv[b][pl.ds(j * 16, 16)]
                    for e in range(16):
                        i = j * 16 + e
                        w = wv[e]
                        for g in range(4):
                            sl = pl.ds(g * 16, 16)
                            rows[b][i, sl] = jnp.maximum(
                                rows[b][i, sl] + emb[b][i, sl], 0.0) * w
                    return 0
                lax.fori_loop(0, _CB // 16, ebody, 0)
                pltpu.sync_copy(rows[b], acc.at[didx[b]], add=True)

            for p in range(npass):
                def zb(i, _):
                    for g in range(4):
                        rows[0][i, pl.ds(g * 16, 16)] = jnp.zeros((16,), F32)
                    return 0
                lax.fori_loop(0, _CB, zb, 0)
                for (off, sz) in pairs_z:
                    pltpu.sync_copy(rows[0].at[pl.ds(0, sz)],
                                    acc.at[pl.ds(s * zz + off, sz)])
                plsc.subcore_barrier()

                issue_loads(0, 0)

                def step(t, _):
                    k0 = 2 * t
                    issue_loads(1, k0 + 1)
                    wait_loads(0, k0)
                    process(0, p)

                    @pl.when(t + 1 < nchunks // 2)
                    def _():
                        issue_loads(0, k0 + 2)
                    wait_loads(1, k0 + 1)
                    process(1, p)
                    return 0
                lax.fori_loop(0, nchunks // 2, step, 0)

                plsc.subcore_barrier()
                for (off, sz) in pairs_w:
                    pltpu.sync_copy(
                        acc.at[pl.ds(s * zw + off, sz)],
                        out_h.at[pl.ds((c * npass + p) * _VAL_R
                                       + s * zw + off, sz)])
                plsc.subcore_barrier()

        phase(hmt_h, emt_h, srct_h, dstt_h, ewt_h, outt_h,
              _EP, _EP // _NS // _CB, 2)
        phase(hmc_h, emc_h, srcc_h, dstc_h, ewc_h, outc_h,
              _SEP, _SEP // _NS // _CB, 3)

    return k(hmt, emt, src_t, dst_t, ew_t, hmc, emc, src_c, dst_c, ew_c)


def _out_pool(ho, agg2, Wo, bo, nw2, seg2, m_valid):
    M = ho.shape[0]
    R = 512
    nb = pl.cdiv(M, R)

    def body(ho_r, agg_r, Wo_r, bo_r, nw_r, seg_r, emb_r, sum_s, cnt_s):
        i = pl.program_id(0)

        @pl.when(i == 0)
        def _():
            sum_s[...] = jnp.zeros((_G, _H), F32)
            cnt_s[...] = jnp.zeros((_G, _H), F32)

        Wof = Wo_r[...]
        aggO = (jnp.dot(agg_r[0], Wof[:_HH, :], preferred_element_type=F32) +
                jnp.dot(agg_r[1], Wof[_HH:, :], preferred_element_type=F32))
        o = jnp.maximum(ho_r[...] + aggO + bo_r[...], 0.0) * nw_r[...]
        sb = seg_r[...]
        gid = lax.broadcasted_iota(I32, (R, _G), 1)
        rid = lax.broadcasted_iota(I32, (R, _G), 0) + i * R
        oh = jnp.where((sb == gid) & (rid < m_valid), 1.0, 0.0).astype(F32)
        dn = (((0,), (0,)), ((), ()))
        sum_s[...] += lax.dot_general(oh, o, dn, preferred_element_type=F32)
        cnt_s[...] += lax.dot_general(oh, jnp.ones((R, _H), F32), dn,
                                      preferred_element_type=F32)

        @pl.when(i == nb - 1)
        def _():
            emb_r[...] = sum_s[...] / jnp.maximum(cnt_s[...], 1.0)

    return pl.pallas_call(
        body,
        grid=(nb,),
        in_specs=[pl.BlockSpec((R, _H), lambda i: (i, 0)),
                  pl.BlockSpec((2, R, _HH), lambda i: (0, i, 0)),
                  pl.BlockSpec((_H, _H), lambda i: (0, 0)),
                  pl.BlockSpec((1, _H), lambda i: (0, 0)),
                  pl.BlockSpec((R, 1), lambda i: (i, 0)),
                  pl.BlockSpec((R, 1), lambda i: (i, 0))],
        out_specs=[pl.BlockSpec((_G, _H), lambda i: (0, 0))],
        out_shape=[_sds((_G, _H), F32)],
        scratch_shapes=[pltpu.VMEM((_G, _H), F32), pltpu.VMEM((_G, _H), F32)],
    )(ho, agg2, Wo, bo.reshape(1, _H), nw2, seg2)[0]


def kernel(x, node_weight, edge_index, edge_attr, edge_weight, batch,
           context_nodes_mapper, target_nodes_mapper, combined_subgraphs,
           subgraphs_edges_mapper, context_edges_mask,
           ctx_W_node, ctx_b_node, ctx_W_edge, ctx_b_edge, ctx_W_msg, ctx_b_msg,
           ctx_W_out, ctx_b_out,
           tgt_W_node, tgt_b_node, tgt_W_edge, tgt_b_edge, tgt_W_msg, tgt_b_msg,
           tgt_W_out, tgt_b_out):
    src_t = jnp.pad(edge_index[0], (0, _EP - _E))
    dst_t = jnp.pad(edge_index[1], (0, _EP - _E))
    ew_t = jnp.pad(edge_weight, (0, _EP - _E))
    ea_t = jnp.pad(edge_attr, ((0, _EP - _E), (0, 0)))
    src_c = jnp.pad(combined_subgraphs[0], (0, _SEP - _SE))
    dst_c = jnp.pad(combined_subgraphs[1], (0, _SEP - _SE))
    sem_p = jnp.pad(subgraphs_edges_mapper, (0, _SEP - _SE))
    maskf = jnp.pad(context_edges_mask.astype(F32), (0, _SEP - _SE))
    cnm_p = jnp.pad(context_nodes_mapper, (0, _CG - _C))
    tnm_p = jnp.pad(target_nodes_mapper, (0, _NG - _N))

    hmc2, hoc, hmt2, hot = _node_tables(
        x, ctx_W_node, ctx_b_node, ctx_W_msg, ctx_W_out,
        tgt_W_node, tgt_b_node, tgt_W_msg, tgt_W_out)
    emt = _edge_tables(ea_t, tgt_W_edge, tgt_b_edge, tgt_W_msg, tgt_b_msg)

    ea_c, ew_c, idx_c, segc, nw_c, hoc_g, segt = _sc_gather(
        cnm_p, tnm_p, sem_p, src_c, edge_weight, maskf, batch, node_weight,
        hoc, edge_attr)

    emc = _edge_tables(ea_c, ctx_W_edge, ctx_b_edge, ctx_W_msg, ctx_b_msg)

    agg_t, agg_c = _sc_msg(
        hmt2.reshape(2 * _TAB_R, _HH), emt.reshape(2 * _EP, _HH),
        src_t, dst_t, ew_t,
        hmc2.reshape(2 * _TAB_R, _HH), emc.reshape(2 * _SEP, _HH),
        idx_c, dst_c, ew_c)

    emb_t = _out_pool(hot, agg_t.reshape(2, 2 * _VAL_R, _HH), tgt_W_out,
                      tgt_b_out, node_weight.reshape(_N, 1),
                      segt[:_N].reshape(_N, 1), _N)
    emb_c = _out_pool(hoc_g, agg_c.reshape(2, 3 * _VAL_R, _HH), ctx_W_out,
                      ctx_b_out, nw_c.reshape(_CG, 1),
                      segc.reshape(_CG, 1), _C)
    return jnp.stack([emb_c, emb_t])


# speedup vs baseline: 1.4752x; 1.1990x over previous
import functools

import jax
import jax.numpy as jnp
from jax import lax
from jax.experimental import pallas as pl
from jax.experimental.pallas import tpu as pltpu
from jax.experimental.pallas import tpu_sc as plsc

F32 = jnp.float32
I32 = jnp.int32

_N = 10000
_E = 320000
_C = 20000
_SE = 100000
_G = 128
_H = 128
_HQ = 32

_NC = 2
_NS = 16
_TAB_R = 81920

_EP = 327680
_SEP = 114688
_CG = 20480
_NG = 10240

_ACC_R = 10496
_VAL_R = 10240
_DUMMY = 10490
_CB = 512


def _sds(shape, dtype):
    return jax.ShapeDtypeStruct(shape, dtype)


def _node_tables(x, cWn, cbn, cWm, cWo, tWn, tbn, tWm, tWo):
    R = 512
    nb = pl.cdiv(_N, R)

    def body(x_r, cWn_r, cbn_r, cWm_r, cWo_r, tWn_r, tbn_r, tWm_r, tWo_r,
             hmc_r, hoc_r, hmt_r, hot_r):
        xb = x_r[...]
        hc = jnp.maximum(
            jnp.dot(xb, cWn_r[...], preferred_element_type=F32) + cbn_r[...], 0.0)
        hmc = jnp.dot(hc, cWm_r[...], preferred_element_type=F32)
        for g in range(4):
            hmc_r[g] = hmc[:, g * _HQ:(g + 1) * _HQ]
        hoc_r[...] = jnp.dot(hc, cWo_r[...], preferred_element_type=F32)
        ht = jnp.maximum(
            jnp.dot(xb, tWn_r[...], preferred_element_type=F32) + tbn_r[...], 0.0)
        hmt = jnp.dot(ht, tWm_r[...], preferred_element_type=F32)
        for g in range(4):
            hmt_r[g] = hmt[:, g * _HQ:(g + 1) * _HQ]
        hot_r[...] = jnp.dot(ht, tWo_r[...], preferred_element_type=F32)

    wsp = pl.BlockSpec((_H, _H), lambda i: (0, 0))
    bsp = pl.BlockSpec((1, _H), lambda i: (0, 0))
    return pl.pallas_call(
        body,
        grid=(nb,),
        in_specs=[pl.BlockSpec((R, _H), lambda i: (i, 0)),
                  wsp, bsp, wsp, wsp, wsp, bsp, wsp, wsp],
        out_specs=[pl.BlockSpec((4, R, _HQ), lambda i: (0, i, 0)),
                   pl.BlockSpec((R, _H), lambda i: (i, 0)),
                   pl.BlockSpec((4, R, _HQ), lambda i: (0, i, 0)),
                   pl.BlockSpec((R, _H), lambda i: (i, 0))],
        out_shape=[_sds((4, _TAB_R, _HQ), F32), _sds((_N, _H), F32),
                   _sds((4, _TAB_R, _HQ), F32), _sds((_N, _H), F32)],
    )(x, cWn, cbn.reshape(1, _H), cWm, cWo, tWn, tbn.reshape(1, _H), tWm, tWo)


def _edge_tables(ea_pad, We, be, Wm, bm):
    M = ea_pad.shape[0]
    R = 1024
    nb = M // R

    def body(ea_r, We_r, be_r, Wm_r, bm_r, em_r):
        e = jnp.maximum(
            jnp.dot(ea_r[...], We_r[...], preferred_element_type=F32) + be_r[...],
            0.0)
        em = jnp.dot(e, Wm_r[...], preferred_element_type=F32) + bm_r[...]
        for g in range(4):
            em_r[g] = em[:, g * _HQ:(g + 1) * _HQ]

    return pl.pallas_call(
        body,
        grid=(nb,),
        in_specs=[pl.BlockSpec((R, 16), lambda i: (i, 0)),
                  pl.BlockSpec((16, _H), lambda i: (0, 0)),
                  pl.BlockSpec((1, _H), lambda i: (0, 0)),
                  pl.BlockSpec((_H, _H), lambda i: (0, 0)),
                  pl.BlockSpec((1, _H), lambda i: (0, 0))],
        out_specs=[pl.BlockSpec((4, R, _HQ), lambda i: (0, i, 0))],
        out_shape=[_sds((4, M, _HQ), F32)],
    )(ea_pad, We, be.reshape(1, _H), Wm, bm.reshape(1, _H))[0]


def _sc_gather(cnm_pad, tnm_pad, sem_pad, srcc_pad, ew, maskf_pad, batch, nw,
               hoc, ea):
    mesh = plsc.VectorSubcoreMesh(core_axis_name="c", subcore_axis_name="s",
                                  num_cores=_NC, num_subcores=_NS)

    @functools.partial(
        pl.kernel,
        out_type=[_sds((_SEP, 16), F32), _sds((_SEP,), F32),
                  _sds((_SEP,), I32),
                  _sds((_CG,), I32), _sds((_CG,), F32),
                  _sds((_CG, _H), F32), _sds((_NG,), I32)],
        mesh=mesh,
        compiler_params=pltpu.CompilerParams(use_tc_tiling_on_sc=False),
        scratch_types=[
            pltpu.VMEM((512,), I32),
            pltpu.VMEM((512,), I32),
            pltpu.VMEM((512,), I32),
            pltpu.VMEM((512, 16), F32),
            pltpu.VMEM((512,), F32),
            pltpu.VMEM((512,), F32),
            pltpu.VMEM((320,), I32),
            pltpu.VMEM((320, _H), F32),
            pltpu.VMEM((320,), I32),
            pltpu.VMEM((320,), F32),
            pltpu.SemaphoreType.DMA,
        ],
    )
    def k(cnm_h, tnm_h, sem_h, srcc_h, ew_h, maskf_h, batch_h, nw_h, hoc_h,
          ea_h,
          eac_o, ewc_o, idxc_o, segc_o, nwc_o, hocg_o, segt_o,
          semb, srcb, cmpb, eab, ewb, maskb, idxb, hocb, segb, nwb, dsem):
        wid = lax.axis_index("s") * _NC + lax.axis_index("c")

        for kk in range(7):
            base = (wid * 7 + kk) * 512
            pltpu.sync_copy(sem_h.at[pl.ds(base, 512)], semb)
            pltpu.sync_copy(srcc_h.at[pl.ds(base, 512)], srcb)
            pltpu.async_copy(ea_h.at[semb], eab, dsem).wait()
            pltpu.sync_copy(eab, eac_o.at[pl.ds(base, 512)])
            pltpu.async_copy(cnm_h.at[srcb], cmpb, dsem).wait()
            pltpu.sync_copy(cmpb, idxc_o.at[pl.ds(base, 512)])
            pltpu.async_copy(ew_h.at[semb], ewb, dsem).wait()
            pltpu.sync_copy(maskf_h.at[pl.ds(base, 512)], maskb)
            for j in range(32):
                sl = pl.ds(j * 16, 16)
                ewb[sl] = ewb[sl] * maskb[sl]
            pltpu.sync_copy(ewb, ewc_o.at[pl.ds(base, 512)])

        for kk in range(2):
            base = (wid * 2 + kk) * 320
            pltpu.sync_copy(cnm_h.at[pl.ds(base, 320)], idxb)
            pltpu.async_copy(hoc_h.at[idxb], hocb, dsem).wait()
            pltpu.sync_copy(hocb, hocg_o.at[pl.ds(base, 320)])
            pltpu.async_copy(batch_h.at[idxb], segb, dsem).wait()
            pltpu.async_copy(nw_h.at[idxb], nwb, dsem).wait()
            pltpu.sync_copy(segb, segc_o.at[pl.ds(base, 320)])
            pltpu.sync_copy(nwb, nwc_o.at[pl.ds(base, 320)])

        base = wid * 320
        pltpu.sync_copy(tnm_h.at[pl.ds(base, 320)], idxb)
        pltpu.async_copy(batch_h.at[idxb], segb, dsem).wait()
        pltpu.sync_copy(segb, segt_o.at[pl.ds(base, 320)])

    return k(cnm_pad, tnm_pad, sem_pad, srcc_pad, ew, maskf_pad, batch, nw,
             hoc, ea)


def _sc_msg(hmt, emt, src_t, dst_t, ew_t, hmc, emc, src_c, dst_c, ew_c):
    mesh = plsc.VectorSubcoreMesh(core_axis_name="c", subcore_axis_name="s",
                                  num_cores=_NC, num_subcores=_NS)
    zz = _ACC_R // _NS
    zw = _VAL_R // _NS

    def _pairs(total):
        out, o = [], 0
        while o < total:
            sz = min(_CB, total - o)
            out.append((o, sz))
            o += sz
        return out

    pairs_z = _pairs(zz)
    pairs_w = _pairs(zw)

    nset = 2
    scratch = [
        [pltpu.VMEM((_CB,), I32) for _ in range(nset)],
        [pltpu.VMEM((_CB,), I32) for _ in range(nset)],
        [pltpu.VMEM((_CB,), F32) for _ in range(nset)],
        [pltpu.VMEM((_CB, _HQ), F32) for _ in range(nset)],
        [pltpu.VMEM((_CB, _HQ), F32) for _ in range(nset)],
        [pltpu.SemaphoreType.DMA for _ in range(nset)],
        [pltpu.SemaphoreType.DMA for _ in range(nset)],
        pltpu.VMEM_SHARED((_ACC_R, _HQ), F32),
    ]

    @functools.partial(
        pl.kernel,
        out_type=[_sds((4 * _VAL_R, _HQ), F32),
                  _sds((8 * _VAL_R, _HQ), F32)],
        mesh=mesh, scratch_types=scratch,
        compiler_params=pltpu.CompilerParams(use_tc_tiling_on_sc=False))
    def k(hmt_h, emt_h, srct_h, dstt_h, ewt_h,
          hmc_h, emc_h, srcc_h, dstc_h, ewc_h,
          outt_h, outc_h,
          sidx, didx, ewbv, rows, emb, lsem, gsem, acc):
        c = lax.axis_index("c")
        s = lax.axis_index("s")

        def phase(hm_h, em_h, src_h, dst_h, ew_h, out_h, ep, nchunks, npass):
            ec = ep // _NS

            def sweep(g, p):
                def issue_loads(b, kk):
                    base = s * ec + kk * _CB
                    pltpu.async_copy(src_h.at[pl.ds(base, _CB)], sidx[b],
                                     lsem[b])
                    pltpu.async_copy(dst_h.at[pl.ds(base, _CB)], didx[b],
                                     lsem[b])
                    pltpu.async_copy(ew_h.at[pl.ds(base, _CB)], ewbv[b],
                                     lsem[b])
                    pltpu.async_copy(em_h.at[pl.ds(g * ep + base, _CB)],
                                     emb[b], lsem[b])

                def wait_loads(b, kk):
                    base = s * ec + kk * _CB
                    pltpu.make_async_copy(src_h.at[pl.ds(base, _CB)], sidx[b],
                                          lsem[b]).wait()
                    pltpu.make_async_copy(dst_h.at[pl.ds(base, _CB)], didx[b],
                                          lsem[b]).wait()
                    pltpu.make_async_copy(ew_h.at[pl.ds(base, _CB)], ewbv[b],
                                          lsem[b]).wait()
                    pltpu.make_async_copy(em_h.at[pl.ds(g * ep + base, _CB)],
                                          emb[b], lsem[b]).wait()

                def process(b):
                    goff = g * _TAB_R
                    for j in range(_CB // 16):
                        sl = pl.ds(j * 16, 16)
                        sidx[b][sl] = sidx[b][sl] + goff
                    gd = pltpu.async_copy(hm_h.at[sidx[b]], rows[b], gsem[b])
                    if npass > 1:
                        for j in range(_CB // 16):
                            sl = pl.ds(j * 16, 16)
                            rel = didx[b][sl] - p * _VAL_R
                            ok = (rel >= 0) & (rel < _VAL_R)
                            didx[b][sl] = jnp.where(ok, rel, _DUMMY)
                    gd.wait()

                    def ebody(j, _):
                        wv = ewbv[b][pl.ds(j * 16, 16)]
                        for e in range(16):
                            i = j * 16 + e
                            w = wv[e]
                            for q in range(2):
                                sl = pl.ds(q * 16, 16)
                                rows[b][i, sl] = jnp.maximum(
                                    rows[b][i, sl] + emb[b][i, sl], 0.0) * w
                        return 0
                    lax.fori_loop(0, _CB // 16, ebody, 0)
                    pltpu.sync_copy(rows[b], acc.at[didx[b]], add=True)

                def zb(i, _):
                    for q in range(2):
                        rows[0][i, pl.ds(q * 16, 16)] = jnp.zeros((16,), F32)
                    return 0
                lax.fori_loop(0, _CB, zb, 0)
                for (off, sz) in pairs_z:
                    pltpu.sync_copy(rows[0].at[pl.ds(0, sz)],
                                    acc.at[pl.ds(s * zz + off, sz)])
                plsc.subcore_barrier()

                issue_loads(0, 0)

                def step(t, _):
                    k0 = 2 * t
                    issue_loads(1, k0 + 1)
                    wait_loads(0, k0)
                    process(0)

                    @pl.when(t + 1 < nchunks // 2)
                    def _():
                        issue_loads(0, k0 + 2)
                    wait_loads(1, k0 + 1)
                    process(1)
                    return 0
                lax.fori_loop(0, nchunks // 2, step, 0)

                plsc.subcore_barrier()
                for (off, sz) in pairs_w:
                    pltpu.sync_copy(
                        acc.at[pl.ds(s * zw + off, sz)],
                        out_h.at[pl.ds((g * npass + p) * _VAL_R
                                       + s * zw + off, sz)])
                plsc.subcore_barrier()

            for gi in range(2):
                g = c * 2 + gi
                for p in range(npass):
                    sweep(g, p)

        phase(hmt_h, emt_h, srct_h, dstt_h, ewt_h, outt_h,
              _EP, _EP // _NS // _CB, 1)
        phase(hmc_h, emc_h, srcc_h, dstc_h, ewc_h, outc_h,
              _SEP, _SEP // _NS // _CB, 2)

    return k(hmt, emt, src_t, dst_t, ew_t, hmc, emc, src_c, dst_c, ew_c)


def _out_pool(ho, agg4, Wo, bo, nw2, seg2, m_valid):
    M = ho.shape[0]
    R = 512
    nb = pl.cdiv(M, R)

    def body(ho_r, agg_r, Wo_r, bo_r, nw_r, seg_r, emb_r, sum_s, cnt_s):
        i = pl.program_id(0)

        @pl.when(i == 0)
        def _():
            sum_s[...] = jnp.zeros((_G, _H), F32)
            cnt_s[...] = jnp.zeros((_G, _H), F32)

        Wof = Wo_r[...]
        aggO = jnp.dot(agg_r[0], Wof[:_HQ, :], preferred_element_type=F32)
        for g in range(1, 4):
            aggO += jnp.dot(agg_r[g], Wof[g * _HQ:(g + 1) * _HQ, :],
                            preferred_element_type=F32)
        o = jnp.maximum(ho_r[...] + aggO + bo_r[...], 0.0) * nw_r[...]
        sb = seg_r[...]
        gid = lax.broadcasted_iota(I32, (R, _G), 1)
        rid = lax.broadcasted_iota(I32, (R, _G), 0) + i * R
        oh = jnp.where((sb == gid) & (rid < m_valid), 1.0, 0.0).astype(F32)
        dn = (((0,), (0,)), ((), ()))
        sum_s[...] += lax.dot_general(oh, o, dn, preferred_element_type=F32)
        cnt_s[...] += lax.dot_general(oh, jnp.ones((R, _H), F32), dn,
                                      preferred_element_type=F32)

        @pl.when(i == nb - 1)
        def _():
            emb_r[...] = sum_s[...] / jnp.maximum(cnt_s[...], 1.0)

    return pl.pallas_call(
        body,
        grid=(nb,),
        in_specs=[pl.BlockSpec((R, _H), lambda i: (i, 0)),
                  pl.BlockSpec((4, R, _HQ), lambda i: (0, i, 0)),
                  pl.BlockSpec((_H, _H), lambda i: (0, 0)),
                  pl.BlockSpec((1, _H), lambda i: (0, 0)),
                  pl.BlockSpec((R, 1), lambda i: (i, 0)),
                  pl.BlockSpec((R, 1), lambda i: (i, 0))],
        out_specs=[pl.BlockSpec((_G, _H), lambda i: (0, 0))],
        out_shape=[_sds((_G, _H), F32)],
        scratch_shapes=[pltpu.VMEM((_G, _H), F32), pltpu.VMEM((_G, _H), F32)],
    )(ho, agg4, Wo, bo.reshape(1, _H), nw2, seg2)[0]


def kernel(x, node_weight, edge_index, edge_attr, edge_weight, batch,
           context_nodes_mapper, target_nodes_mapper, combined_subgraphs,
           subgraphs_edges_mapper, context_edges_mask,
           ctx_W_node, ctx_b_node, ctx_W_edge, ctx_b_edge, ctx_W_msg, ctx_b_msg,
           ctx_W_out, ctx_b_out,
           tgt_W_node, tgt_b_node, tgt_W_edge, tgt_b_edge, tgt_W_msg, tgt_b_msg,
           tgt_W_out, tgt_b_out):
    src_t = jnp.pad(edge_index[0], (0, _EP - _E))
    dst_t = jnp.pad(edge_index[1], (0, _EP - _E))
    ew_t = jnp.pad(edge_weight, (0, _EP - _E))
    ea_t = jnp.pad(edge_attr, ((0, _EP - _E), (0, 0)))
    src_c = jnp.pad(combined_subgraphs[0], (0, _SEP - _SE))
    dst_c = jnp.pad(combined_subgraphs[1], (0, _SEP - _SE))
    sem_p = jnp.pad(subgraphs_edges_mapper, (0, _SEP - _SE))
    maskf = jnp.pad(context_edges_mask.astype(F32), (0, _SEP - _SE))
    cnm_p = jnp.pad(context_nodes_mapper, (0, _CG - _C))
    tnm_p = jnp.pad(target_nodes_mapper, (0, _NG - _N))

    hmc4, hoc, hmt4, hot = _node_tables(
        x, ctx_W_node, ctx_b_node, ctx_W_msg, ctx_W_out,
        tgt_W_node, tgt_b_node, tgt_W_msg, tgt_W_out)
    emt = _edge_tables(ea_t, tgt_W_edge, tgt_b_edge, tgt_W_msg, tgt_b_msg)

    ea_c, ew_c, idx_c, segc, nw_c, hoc_g, segt = _sc_gather(
        cnm_p, tnm_p, sem_p, src_c, edge_weight, maskf, batch, node_weight,
        hoc, edge_attr)

    emc = _edge_tables(ea_c, ctx_W_edge, ctx_b_edge, ctx_W_msg, ctx_b_msg)

    agg_t, agg_c = _sc_msg(
        hmt4.reshape(4 * _TAB_R, _HQ), emt.reshape(4 * _EP, _HQ),
        src_t, dst_t, ew_t,
        hmc4.reshape(4 * _TAB_R, _HQ), emc.reshape(4 * _SEP, _HQ),
        idx_c, dst_c, ew_c)

    emb_t = _out_pool(hot, agg_t.reshape(4, _VAL_R, _HQ), tgt_W_out,
                      tgt_b_out, node_weight.reshape(_N, 1),
                      segt[:_N].reshape(_N, 1), _N)
    emb_c = _out_pool(hoc_g, agg_c.reshape(4, 2 * _VAL_R, _HQ), ctx_W_out,
                      ctx_b_out, nw_c.reshape(_CG, 1),
                      segc.reshape(_CG, 1), _C)
    return jnp.stack([emb_c, emb_t])
